# Initial kernel scaffold; baseline (speedup 1.0000x reference)
import jax
import jax.numpy as jnp
from jax.experimental import pallas as pl


def kernel(x, target):
    raise NotImplementedError("write your pallas kernel here")


# speedup vs baseline: 1.6096x; 1.6096x over previous
import math

import jax
import jax.numpy as jnp
from jax.experimental import pallas as pl

_SMOOTHING = 0.1
_CONFIDENCE = 1.0 - _SMOOTHING
_PAD = 0
_BLOCK_W = 4096


def _make_kernel(batch, v, block_w, eps, c1):
    conf_minus_eps = _CONFIDENCE - eps

    def body(target_ref, x_ref, out_ref):
        j = pl.program_id(0)
        col0 = j * block_w
        cols = jax.lax.broadcasted_iota(jnp.int32, (batch, block_w), 1) + col0
        t = target_ref[:, :]
        valid = t != _PAD
        inb = cols < v
        xv = jnp.where(inb, x_ref[:, :], 0.0)
        w = jnp.where((cols != _PAD) & inb & valid, eps, 0.0)
        w = w + jnp.where((cols == t) & valid, conf_minus_eps, 0.0)
        contrib = jnp.sum(w * xv, keepdims=True)[:1, :1]

        @pl.when(j == 0)
        def _init():
            nvalid = jnp.sum(valid.astype(jnp.float32), keepdims=True)[:1, :1]
            out_ref[:, :] = nvalid * c1

        out_ref[:, :] = out_ref[:, :] - contrib

    return body


def kernel(x, target):
    batch, v = x.shape
    eps = _SMOOTHING / (v - 2)
    c1 = eps * math.log(eps) * (v - 2) + _CONFIDENCE * math.log(_CONFIDENCE)
    num_blocks = (v + _BLOCK_W - 1) // _BLOCK_W
    t2 = target.reshape(batch, 1).astype(jnp.int32)

    out = pl.pallas_call(
        _make_kernel(batch, v, _BLOCK_W, eps, c1),
        grid=(num_blocks,),
        in_specs=[
            pl.BlockSpec((batch, 1), lambda j: (0, 0)),
            pl.BlockSpec((batch, _BLOCK_W), lambda j: (0, j)),
        ],
        out_specs=pl.BlockSpec((1, 1), lambda j: (0, 0)),
        out_shape=jax.ShapeDtypeStruct((1, 1), jnp.float32),
    )(t2, x)
    return out[0, 0]
